# baseline (speedup 1.0000x reference)
import functools

import jax
import jax.numpy as jnp
from jax import lax
from jax.experimental import pallas as pl
from jax.experimental.pallas import tpu as pltpu
from jax.experimental.pallas import tpu_sc as plsc

_NC = 2
_NS = 16


def _sc_gather(x, idx3):
    nw, n_chunks, chunk = idx3.shape
    d = x.shape[1]
    total = nw * n_chunks * chunk
    per_w = n_chunks * chunk
    mesh = plsc.VectorSubcoreMesh(core_axis_name="c", subcore_axis_name="s")

    @functools.partial(
        pl.kernel,
        out_type=jax.ShapeDtypeStruct((total, d), x.dtype),
        mesh=mesh,
        scratch_types=[
            pltpu.VMEM((n_chunks, chunk), jnp.int32),
            pltpu.VMEM((chunk, d), x.dtype),
            pltpu.SemaphoreType.DMA,
        ],
    )
    def k(x_hbm, idx_hbm, out_hbm, idx_v, buf_v, sem):
        wid = lax.axis_index("s") * _NC + lax.axis_index("c")
        base = wid * per_w
        pltpu.sync_copy(idx_hbm.at[wid], idx_v)

        @pl.loop(0, n_chunks)
        def _(j):
            pltpu.async_copy(x_hbm.at[idx_v.at[j]], buf_v, sem).wait()
            pltpu.sync_copy(buf_v, out_hbm.at[pl.ds(base + j * chunk, chunk)])

    return k(x, idx3)


def _sc_scatter_residual(h_t, dst3, x):
    n, d = x.shape
    dh = d // _NC
    ns, n_chunks, chunk = dst3.shape
    per_s = n_chunks * chunk
    rows_main = (n // ns) // 8 * 8
    tail_base = ns * rows_main
    tail_rows = n - tail_base
    mesh = plsc.VectorSubcoreMesh(core_axis_name="c", subcore_axis_name="s")

    @functools.partial(
        pl.kernel,
        out_type=jax.ShapeDtypeStruct((n, d), x.dtype),
        mesh=mesh,
        scratch_types=[
            pltpu.VMEM((n_chunks, chunk), jnp.int32),
            pltpu.VMEM((chunk, dh), x.dtype),
            pltpu.VMEM_SHARED((n, dh), x.dtype),
        ],
    )
    def k(h_hbm, dst_hbm, x_hbm, out_hbm, idx_v, buf_v, acc_sh):
        c = lax.axis_index("c")
        s = lax.axis_index("s")
        col0 = c * dh
        r0 = s * rows_main
        pltpu.sync_copy(
            x_hbm.at[pl.ds(r0, rows_main), pl.ds(col0, dh)],
            acc_sh.at[pl.ds(r0, rows_main)],
        )
        if tail_rows:
            @pl.when(s == ns - 1)
            def _():
                pltpu.sync_copy(
                    x_hbm.at[pl.ds(tail_base, tail_rows), pl.ds(col0, dh)],
                    acc_sh.at[pl.ds(tail_base, tail_rows)],
                )
        pltpu.sync_copy(dst_hbm.at[s], idx_v)
        plsc.subcore_barrier()

        @pl.loop(0, n_chunks)
        def _(j):
            rbase = s * per_s + j * chunk
            pltpu.sync_copy(h_hbm.at[pl.ds(rbase, chunk), pl.ds(col0, dh)], buf_v)
            pltpu.sync_copy(buf_v, acc_sh.at[idx_v.at[j]], add=True)

        plsc.subcore_barrier()
        pltpu.sync_copy(
            acc_sh.at[pl.ds(r0, rows_main)],
            out_hbm.at[pl.ds(r0, rows_main), pl.ds(col0, dh)],
        )
        if tail_rows:
            @pl.when(s == ns - 1)
            def _():
                pltpu.sync_copy(
                    acc_sh.at[pl.ds(tail_base, tail_rows)],
                    out_hbm.at[pl.ds(tail_base, tail_rows), pl.ds(col0, dh)],
                )

    return k(h_t, dst3, x)


def _tc_lstm(seq, w_ih, w_hh, bias, blk):
    p, ld = seq.shape
    g, d = w_ih.shape
    steps = ld // d
    prec = lax.Precision.DEFAULT
    dn = (((1,), (1,)), ((), ()))

    def body(seq_ref, wih_ref, whh_ref, b_ref, out_ref):
        wih = wih_ref[...].astype(jnp.bfloat16)
        whh = whh_ref[...].astype(jnp.bfloat16)
        b = b_ref[...]
        s = seq_ref[...].astype(jnp.bfloat16)
        h = None
        c = None
        for t in range(steps):
            st = s[:, t * d:(t + 1) * d]
            gates = lax.dot_general(st, wih, dn, precision=prec,
                                    preferred_element_type=jnp.float32) + b
            if h is not None:
                gates = gates + lax.dot_general(h.astype(jnp.bfloat16), whh, dn,
                                                precision=prec,
                                                preferred_element_type=jnp.float32)
            gi = jax.nn.sigmoid(gates[:, 0 * d:1 * d])
            gf = jax.nn.sigmoid(gates[:, 1 * d:2 * d])
            gg = jnp.tanh(gates[:, 2 * d:3 * d])
            go = jax.nn.sigmoid(gates[:, 3 * d:4 * d])
            c = gi * gg if c is None else gf * c + gi * gg
            h = go * jnp.tanh(c)
        out_ref[...] = h

    return pl.pallas_call(
        body,
        grid=(p // blk,),
        in_specs=[
            pl.BlockSpec((blk, ld), lambda i: (i, 0)),
            pl.BlockSpec((g, d), lambda i: (0, 0)),
            pl.BlockSpec((g, d), lambda i: (0, 0)),
            pl.BlockSpec((1, g), lambda i: (0, 0)),
        ],
        out_specs=pl.BlockSpec((blk, d), lambda i: (i, 0)),
        out_shape=jax.ShapeDtypeStruct((p, d), seq.dtype),
    )(seq, w_ih, w_hh, bias)


def _tc_bn_relu(y, gamma, beta):
    n, d = y.shape

    def body(y_ref, g_ref, b_ref, o_ref):
        v = y_ref[...]
        mean = jnp.mean(v, axis=0, keepdims=True)
        cent = v - mean
        var = jnp.mean(cent * cent, axis=0, keepdims=True)
        scaled = cent * lax.rsqrt(var + 1e-5) * g_ref[...] + b_ref[...]
        o_ref[...] = jnp.maximum(scaled, 0.0)

    return pl.pallas_call(
        body,
        out_shape=jax.ShapeDtypeStruct((n, d), y.dtype),
    )(y, gamma.reshape(1, d), beta.reshape(1, d))


def kernel(x, paths, W_ih, W_hh, b_ih, b_hh, gamma, beta):
    n, d = x.shape
    p, l = paths.shape
    paths = paths.astype(jnp.int32)
    bias = (b_ih + b_hh).reshape(1, 4 * d).astype(jnp.float32)

    nw = _NC * _NS
    chunk = 80
    per_w = (p * l) // nw
    idx3 = paths.reshape(nw, per_w // chunk, chunk)
    seq = _sc_gather(x, idx3)
    seq = seq.reshape(p, l * d)

    h_t = _tc_lstm(seq, W_ih, W_hh, bias, blk=1000)

    chunk2 = 80
    per_s = p // _NS
    dst3 = paths[:, l - 1].reshape(_NS, per_s // chunk2, chunk2)
    y = _sc_scatter_residual(h_t, dst3, x)

    return _tc_bn_relu(y, gamma, beta)


# speedup vs baseline: 1.1685x; 1.1685x over previous
import functools

import jax
import jax.numpy as jnp
from jax import lax
from jax.experimental import pallas as pl
from jax.experimental.pallas import tpu as pltpu
from jax.experimental.pallas import tpu_sc as plsc

_NC = 2
_NS = 16


def _sc_gather(x, idx3):
    nw, n_chunks, chunk = idx3.shape
    d = x.shape[1]
    total = nw * n_chunks * chunk
    per_w = n_chunks * chunk
    mesh = plsc.VectorSubcoreMesh(core_axis_name="c", subcore_axis_name="s")

    @functools.partial(
        pl.kernel,
        out_type=jax.ShapeDtypeStruct((total, d), x.dtype),
        mesh=mesh,
        scratch_types=[
            pltpu.VMEM((n_chunks, chunk), jnp.int32),
            pltpu.VMEM((chunk, d), x.dtype),
            pltpu.SemaphoreType.DMA,
        ],
    )
    def k(x_hbm, idx_hbm, out_hbm, idx_v, buf_v, sem):
        wid = lax.axis_index("s") * _NC + lax.axis_index("c")
        base = wid * per_w
        pltpu.sync_copy(idx_hbm.at[wid], idx_v)

        @pl.loop(0, n_chunks)
        def _(j):
            pltpu.async_copy(x_hbm.at[idx_v.at[j]], buf_v, sem).wait()
            pltpu.sync_copy(buf_v, out_hbm.at[pl.ds(base + j * chunk, chunk)])

    return k(x, idx3)


def _sc_scatter_residual(h_t, dst3, x):
    n, d = x.shape
    dh = d // _NC
    ns, n_chunks, chunk = dst3.shape
    per_s = n_chunks * chunk
    rows_main = (n // ns) // 8 * 8
    tail_base = ns * rows_main
    tail_rows = n - tail_base
    mesh = plsc.VectorSubcoreMesh(core_axis_name="c", subcore_axis_name="s")

    @functools.partial(
        pl.kernel,
        out_type=jax.ShapeDtypeStruct((n, d), x.dtype),
        mesh=mesh,
        scratch_types=[
            pltpu.VMEM((n_chunks, chunk), jnp.int32),
            pltpu.VMEM((chunk, dh), x.dtype),
            pltpu.VMEM_SHARED((n, dh), x.dtype),
        ],
    )
    def k(h_hbm, dst_hbm, x_hbm, out_hbm, idx_v, buf_v, acc_sh):
        c = lax.axis_index("c")
        s = lax.axis_index("s")
        col0 = c * dh
        r0 = s * rows_main
        pltpu.sync_copy(
            x_hbm.at[pl.ds(r0, rows_main), pl.ds(col0, dh)],
            acc_sh.at[pl.ds(r0, rows_main)],
        )
        if tail_rows:
            @pl.when(s == ns - 1)
            def _():
                pltpu.sync_copy(
                    x_hbm.at[pl.ds(tail_base, tail_rows), pl.ds(col0, dh)],
                    acc_sh.at[pl.ds(tail_base, tail_rows)],
                )
        pltpu.sync_copy(dst_hbm.at[s], idx_v)
        plsc.subcore_barrier()

        @pl.loop(0, n_chunks)
        def _(j):
            rbase = s * per_s + j * chunk
            pltpu.sync_copy(h_hbm.at[pl.ds(rbase, chunk), pl.ds(col0, dh)], buf_v)
            pltpu.sync_copy(buf_v, acc_sh.at[idx_v.at[j]], add=True)

        plsc.subcore_barrier()
        pltpu.sync_copy(
            acc_sh.at[pl.ds(r0, rows_main)],
            out_hbm.at[pl.ds(r0, rows_main), pl.ds(col0, dh)],
        )
        if tail_rows:
            @pl.when(s == ns - 1)
            def _():
                pltpu.sync_copy(
                    acc_sh.at[pl.ds(tail_base, tail_rows)],
                    out_hbm.at[pl.ds(tail_base, tail_rows), pl.ds(col0, dh)],
                )

    return k(h_t, dst3, x)


def _tc_lstm(seq, w_ih, w_hh, bias, blk):
    p, ld = seq.shape
    g, d = w_ih.shape
    steps = ld // d
    prec = lax.Precision.DEFAULT
    dn = (((1,), (1,)), ((), ()))

    def body(seq_ref, wih_ref, whh_ref, b_ref, out_ref):
        wih = wih_ref[...]
        whh = whh_ref[...]
        b = b_ref[...]
        s = seq_ref[...]
        h = None
        c = None
        for t in range(steps):
            st = s[:, t * d:(t + 1) * d]
            gates = lax.dot_general(st, wih, dn, precision=prec,
                                    preferred_element_type=jnp.float32) + b
            if h is not None:
                gates = gates + lax.dot_general(h, whh, dn, precision=prec,
                                                preferred_element_type=jnp.float32)
            gi = jax.nn.sigmoid(gates[:, 0 * d:1 * d])
            gf = jax.nn.sigmoid(gates[:, 1 * d:2 * d])
            gg = jnp.tanh(gates[:, 2 * d:3 * d])
            go = jax.nn.sigmoid(gates[:, 3 * d:4 * d])
            c = gi * gg if c is None else gf * c + gi * gg
            h = go * jnp.tanh(c)
        out_ref[...] = h

    return pl.pallas_call(
        body,
        grid=(p // blk,),
        in_specs=[
            pl.BlockSpec((blk, ld), lambda i: (i, 0)),
            pl.BlockSpec((g, d), lambda i: (0, 0)),
            pl.BlockSpec((g, d), lambda i: (0, 0)),
            pl.BlockSpec((1, g), lambda i: (0, 0)),
        ],
        out_specs=pl.BlockSpec((blk, d), lambda i: (i, 0)),
        out_shape=jax.ShapeDtypeStruct((p, d), seq.dtype),
    )(seq, w_ih, w_hh, bias)


def _tc_bn_relu(y, gamma, beta):
    n, d = y.shape

    def body(y_ref, g_ref, b_ref, o_ref):
        v = y_ref[...]
        mean = jnp.mean(v, axis=0, keepdims=True)
        cent = v - mean
        var = jnp.mean(cent * cent, axis=0, keepdims=True)
        scaled = cent * lax.rsqrt(var + 1e-5) * g_ref[...] + b_ref[...]
        o_ref[...] = jnp.maximum(scaled, 0.0)

    return pl.pallas_call(
        body,
        out_shape=jax.ShapeDtypeStruct((n, d), y.dtype),
    )(y, gamma.reshape(1, d), beta.reshape(1, d))


def kernel(x, paths, W_ih, W_hh, b_ih, b_hh, gamma, beta):
    n, d = x.shape
    p, l = paths.shape
    paths = paths.astype(jnp.int32)
    bias = (b_ih + b_hh).reshape(1, 4 * d).astype(jnp.float32)

    nw = _NC * _NS
    chunk = 80
    n_slices = 5
    ps = p // n_slices
    per_w = (ps * l) // nw
    h_parts = []
    for k in range(n_slices):
        pk = paths[k * ps:(k + 1) * ps]
        idx3 = pk.reshape(nw, per_w // chunk, chunk)
        seq = _sc_gather(x, idx3).reshape(ps, l * d)
        h_parts.append(_tc_lstm(seq, W_ih, W_hh, bias, blk=1000))
    h_t = jnp.concatenate(h_parts, axis=0)

    chunk2 = 80
    per_s = p // _NS
    dst3 = paths[:, l - 1].reshape(_NS, per_s // chunk2, chunk2)
    y = _sc_scatter_residual(h_t, dst3, x)

    return _tc_bn_relu(y, gamma, beta)


# speedup vs baseline: 1.4573x; 1.2472x over previous
import functools

import jax
import jax.numpy as jnp
from jax import lax
from jax.experimental import pallas as pl
from jax.experimental.pallas import tpu as pltpu
from jax.experimental.pallas import tpu_sc as plsc

_NC = 2
_NS = 16


def _sc_gather(x, idx3):
    nw, n_chunks, chunk = idx3.shape
    d = x.shape[1]
    total = nw * n_chunks * chunk
    per_w = n_chunks * chunk
    mesh = plsc.VectorSubcoreMesh(core_axis_name="c", subcore_axis_name="s")

    @functools.partial(
        pl.kernel,
        out_type=jax.ShapeDtypeStruct((total, d), x.dtype),
        mesh=mesh,
        scratch_types=[
            pltpu.VMEM((n_chunks, chunk), jnp.int32),
            pltpu.VMEM((chunk, d), x.dtype),
            pltpu.SemaphoreType.DMA,
        ],
    )
    def k(x_hbm, idx_hbm, out_hbm, idx_v, buf_v, sem):
        wid = lax.axis_index("s") * _NC + lax.axis_index("c")
        base = wid * per_w
        pltpu.sync_copy(idx_hbm.at[wid], idx_v)

        @pl.loop(0, n_chunks)
        def _(j):
            pltpu.async_copy(x_hbm.at[idx_v.at[j]], buf_v, sem).wait()
            pltpu.sync_copy(buf_v, out_hbm.at[pl.ds(base + j * chunk, chunk)])

    return k(x, idx3)


def _sc_scatter_residual(h_t, dst3, x):
    n, d = x.shape
    dh = d // _NC
    ns, n_chunks, chunk = dst3.shape
    per_s = n_chunks * chunk
    rows_main = (n // ns) // 8 * 8
    tail_base = ns * rows_main
    tail_rows = n - tail_base
    mesh = plsc.VectorSubcoreMesh(core_axis_name="c", subcore_axis_name="s")

    @functools.partial(
        pl.kernel,
        out_type=jax.ShapeDtypeStruct((n, d), x.dtype),
        mesh=mesh,
        scratch_types=[
            pltpu.VMEM((n_chunks, chunk), jnp.int32),
            pltpu.VMEM((chunk, dh), x.dtype),
            pltpu.VMEM_SHARED((n, dh), x.dtype),
        ],
    )
    def k(h_hbm, dst_hbm, x_hbm, out_hbm, idx_v, buf_v, acc_sh):
        c = lax.axis_index("c")
        s = lax.axis_index("s")
        col0 = c * dh
        r0 = s * rows_main
        pltpu.sync_copy(
            x_hbm.at[pl.ds(r0, rows_main), pl.ds(col0, dh)],
            acc_sh.at[pl.ds(r0, rows_main)],
        )
        if tail_rows:
            @pl.when(s == ns - 1)
            def _():
                pltpu.sync_copy(
                    x_hbm.at[pl.ds(tail_base, tail_rows), pl.ds(col0, dh)],
                    acc_sh.at[pl.ds(tail_base, tail_rows)],
                )
        pltpu.sync_copy(dst_hbm.at[s], idx_v)
        plsc.subcore_barrier()

        @pl.loop(0, n_chunks)
        def _(j):
            rbase = s * per_s + j * chunk
            pltpu.sync_copy(h_hbm.at[pl.ds(rbase, chunk), pl.ds(col0, dh)], buf_v)
            pltpu.sync_copy(buf_v, acc_sh.at[idx_v.at[j]], add=True)

        plsc.subcore_barrier()
        pltpu.sync_copy(
            acc_sh.at[pl.ds(r0, rows_main)],
            out_hbm.at[pl.ds(r0, rows_main), pl.ds(col0, dh)],
        )
        if tail_rows:
            @pl.when(s == ns - 1)
            def _():
                pltpu.sync_copy(
                    acc_sh.at[pl.ds(tail_base, tail_rows)],
                    out_hbm.at[pl.ds(tail_base, tail_rows), pl.ds(col0, dh)],
                )

    return k(h_t, dst3, x)


def _tc_lstm(seq_all, steps, w_ih, w_hh, bias, blk):
    lp, d = seq_all.shape
    g = w_ih.shape[0]
    p = lp // steps
    nblk = p // blk
    prec = lax.Precision.DEFAULT
    dn = (((1,), (1,)), ((), ()))

    def body(*refs):
        s_refs = refs[:steps]
        wih_ref, whh_ref, b_ref, out_ref = refs[steps:]
        wih = wih_ref[...]
        whh = whh_ref[...]
        b = b_ref[...]
        h = None
        c = None
        for t in range(steps):
            st = s_refs[t][...]
            gates = lax.dot_general(st, wih, dn, precision=prec,
                                    preferred_element_type=jnp.float32) + b
            if h is not None:
                gates = gates + lax.dot_general(h, whh, dn, precision=prec,
                                                preferred_element_type=jnp.float32)
            gi = jax.nn.sigmoid(gates[:, 0 * d:1 * d])
            gf = jax.nn.sigmoid(gates[:, 1 * d:2 * d])
            gg = jnp.tanh(gates[:, 2 * d:3 * d])
            go = jax.nn.sigmoid(gates[:, 3 * d:4 * d])
            c = gi * gg if c is None else gf * c + gi * gg
            h = go * jnp.tanh(c)
        out_ref[...] = h

    seq_specs = [
        pl.BlockSpec((blk, d), lambda i, t=t: (t * nblk + i, 0))
        for t in range(steps)
    ]
    return pl.pallas_call(
        body,
        grid=(nblk,),
        in_specs=seq_specs + [
            pl.BlockSpec((g, d), lambda i: (0, 0)),
            pl.BlockSpec((g, d), lambda i: (0, 0)),
            pl.BlockSpec((1, g), lambda i: (0, 0)),
        ],
        out_specs=pl.BlockSpec((blk, d), lambda i: (i, 0)),
        out_shape=jax.ShapeDtypeStruct((p, d), seq_all.dtype),
    )(*([seq_all] * steps), w_ih, w_hh, bias)


def _tc_bn_relu(y, gamma, beta):
    n, d = y.shape

    def body(y_ref, g_ref, b_ref, o_ref):
        v = y_ref[...]
        mean = jnp.mean(v, axis=0, keepdims=True)
        cent = v - mean
        var = jnp.mean(cent * cent, axis=0, keepdims=True)
        scaled = cent * lax.rsqrt(var + 1e-5) * g_ref[...] + b_ref[...]
        o_ref[...] = jnp.maximum(scaled, 0.0)

    return pl.pallas_call(
        body,
        out_shape=jax.ShapeDtypeStruct((n, d), y.dtype),
    )(y, gamma.reshape(1, d), beta.reshape(1, d))


def kernel(x, paths, W_ih, W_hh, b_ih, b_hh, gamma, beta):
    n, d = x.shape
    p, l = paths.shape
    paths = paths.astype(jnp.int32)
    bias = (b_ih + b_hh).reshape(1, 4 * d).astype(jnp.float32)

    nw = _NC * _NS
    chunk = 80
    per_w = (p * l) // nw
    idx3 = paths.T.reshape(nw, per_w // chunk, chunk)
    seq_all = _sc_gather(x, idx3)

    h_t = _tc_lstm(seq_all, l, W_ih, W_hh, bias, blk=1000)

    chunk2 = 80
    per_s = p // _NS
    dst3 = paths[:, l - 1].reshape(_NS, per_s // chunk2, chunk2)
    y = _sc_scatter_residual(h_t, dst3, x)

    return _tc_bn_relu(y, gamma, beta)


# speedup vs baseline: 1.4825x; 1.0173x over previous
import functools

import jax
import jax.numpy as jnp
from jax import lax
from jax.experimental import pallas as pl
from jax.experimental.pallas import tpu as pltpu
from jax.experimental.pallas import tpu_sc as plsc

_NC = 2
_NS = 16


def _sc_gather(x, idx3):
    nw, n_chunks, chunk = idx3.shape
    d = x.shape[1]
    total = nw * n_chunks * chunk
    per_w = n_chunks * chunk
    mesh = plsc.VectorSubcoreMesh(core_axis_name="c", subcore_axis_name="s")

    @functools.partial(
        pl.kernel,
        out_type=jax.ShapeDtypeStruct((total, d), x.dtype),
        mesh=mesh,
        scratch_types=[
            pltpu.VMEM((n_chunks, chunk), jnp.int32),
            pltpu.VMEM((chunk, d), x.dtype),
            pltpu.SemaphoreType.DMA,
        ],
    )
    def k(x_hbm, idx_hbm, out_hbm, idx_v, buf_v, sem):
        wid = lax.axis_index("s") * _NC + lax.axis_index("c")
        base = wid * per_w
        pltpu.sync_copy(idx_hbm.at[wid], idx_v)

        @pl.loop(0, n_chunks)
        def _(j):
            pltpu.async_copy(x_hbm.at[idx_v.at[j]], buf_v, sem).wait()
            pltpu.sync_copy(buf_v, out_hbm.at[pl.ds(base + j * chunk, chunk)])

    return k(x, idx3)


def _sc_scatter_residual(h_t, dst3, x):
    n, d = x.shape
    dh = d // _NC
    ns, n_chunks, chunk = dst3.shape
    per_s = n_chunks * chunk
    rows_main = (n // ns) // 8 * 8
    tail_base = ns * rows_main
    tail_rows = n - tail_base
    mesh = plsc.VectorSubcoreMesh(core_axis_name="c", subcore_axis_name="s")

    @functools.partial(
        pl.kernel,
        out_type=jax.ShapeDtypeStruct((n, d), x.dtype),
        mesh=mesh,
        scratch_types=[
            pltpu.VMEM((n_chunks, chunk), jnp.int32),
            pltpu.VMEM((chunk, dh), x.dtype),
            pltpu.VMEM_SHARED((n, dh), x.dtype),
        ],
    )
    def k(h_hbm, dst_hbm, x_hbm, out_hbm, idx_v, buf_v, acc_sh):
        c = lax.axis_index("c")
        s = lax.axis_index("s")
        col0 = c * dh
        r0 = s * rows_main
        pltpu.sync_copy(
            x_hbm.at[pl.ds(r0, rows_main), pl.ds(col0, dh)],
            acc_sh.at[pl.ds(r0, rows_main)],
        )
        if tail_rows:
            @pl.when(s == ns - 1)
            def _():
                pltpu.sync_copy(
                    x_hbm.at[pl.ds(tail_base, tail_rows), pl.ds(col0, dh)],
                    acc_sh.at[pl.ds(tail_base, tail_rows)],
                )
        pltpu.sync_copy(dst_hbm.at[s], idx_v)
        plsc.subcore_barrier()

        @pl.loop(0, n_chunks)
        def _(j):
            rbase = s * per_s + j * chunk
            pltpu.sync_copy(h_hbm.at[pl.ds(rbase, chunk), pl.ds(col0, dh)], buf_v)
            pltpu.sync_copy(buf_v, acc_sh.at[idx_v.at[j]], add=True)

        plsc.subcore_barrier()
        pltpu.sync_copy(
            acc_sh.at[pl.ds(r0, rows_main)],
            out_hbm.at[pl.ds(r0, rows_main), pl.ds(col0, dh)],
        )
        if tail_rows:
            @pl.when(s == ns - 1)
            def _():
                pltpu.sync_copy(
                    acc_sh.at[pl.ds(tail_base, tail_rows)],
                    out_hbm.at[pl.ds(tail_base, tail_rows), pl.ds(col0, dh)],
                )

    return k(h_t, dst3, x)


def _tc_lstm(seq_all, steps, w_ih, w_hh, bias, blk):
    lp, d = seq_all.shape
    g = w_ih.shape[0]
    p = lp // steps
    nblk = p // blk
    prec = lax.Precision.DEFAULT
    dn = (((1,), (1,)), ((), ()))

    def body(*refs):
        s_refs = refs[:steps]
        wih_ref, whh_ref, b_ref, out_ref = refs[steps:]
        wih = wih_ref[...]
        whh = whh_ref[...]
        b = b_ref[...]
        h = None
        c = None
        for t in range(steps):
            st = s_refs[t][...]
            gates = lax.dot_general(st, wih, dn, precision=prec,
                                    preferred_element_type=jnp.float32) + b
            if h is not None:
                gates = gates + lax.dot_general(h, whh, dn, precision=prec,
                                                preferred_element_type=jnp.float32)
            gi = jax.nn.sigmoid(gates[:, 0 * d:1 * d])
            gf = jax.nn.sigmoid(gates[:, 1 * d:2 * d])
            gg = jnp.tanh(gates[:, 2 * d:3 * d])
            go = jax.nn.sigmoid(gates[:, 3 * d:4 * d])
            c = gi * gg if c is None else gf * c + gi * gg
            h = go * jnp.tanh(c)
        out_ref[...] = h

    seq_specs = [
        pl.BlockSpec((blk, d), lambda i, t=t: (t * nblk + i, 0))
        for t in range(steps)
    ]
    return pl.pallas_call(
        body,
        grid=(nblk,),
        in_specs=seq_specs + [
            pl.BlockSpec((g, d), lambda i: (0, 0)),
            pl.BlockSpec((g, d), lambda i: (0, 0)),
            pl.BlockSpec((1, g), lambda i: (0, 0)),
        ],
        out_specs=pl.BlockSpec((blk, d), lambda i: (i, 0)),
        out_shape=jax.ShapeDtypeStruct((p, d), seq_all.dtype),
    )(*([seq_all] * steps), w_ih, w_hh, bias)


def _tc_bn_relu(y, gamma, beta):
    n, d = y.shape

    def body(y_ref, g_ref, b_ref, o_ref):
        v = y_ref[...]
        mean = jnp.mean(v, axis=0, keepdims=True)
        cent = v - mean
        var = jnp.mean(cent * cent, axis=0, keepdims=True)
        scaled = cent * lax.rsqrt(var + 1e-5) * g_ref[...] + b_ref[...]
        o_ref[...] = jnp.maximum(scaled, 0.0)

    return pl.pallas_call(
        body,
        out_shape=jax.ShapeDtypeStruct((n, d), y.dtype),
    )(y, gamma.reshape(1, d), beta.reshape(1, d))


def kernel(x, paths, W_ih, W_hh, b_ih, b_hh, gamma, beta):
    n, d = x.shape
    p, l = paths.shape
    paths = paths.astype(jnp.int32)
    bias = (b_ih + b_hh).reshape(1, 4 * d).astype(jnp.float32)

    nw = _NC * _NS
    chunk = 80
    per_w = (p * l) // nw
    idx3 = paths.T.reshape(nw, per_w // chunk, chunk)
    seq_all = _sc_gather(x, idx3)

    h_t = _tc_lstm(seq_all, l, W_ih, W_hh, bias, blk=2000)

    chunk2 = 80
    per_s = p // _NS
    dst3 = paths[:, l - 1].reshape(_NS, per_s // chunk2, chunk2)
    y = _sc_scatter_residual(h_t, dst3, x)

    return _tc_bn_relu(y, gamma, beta)


# speedup vs baseline: 1.8064x; 1.2185x over previous
import functools

import jax
import jax.numpy as jnp
from jax import lax
from jax.experimental import pallas as pl
from jax.experimental.pallas import tpu as pltpu
from jax.experimental.pallas import tpu_sc as plsc

_NC = 2
_NS = 16


def _sc_gather(x, idx3):
    nw, n_chunks, chunk = idx3.shape
    d = x.shape[1]
    total = nw * n_chunks * chunk
    per_w = n_chunks * chunk
    mesh = plsc.VectorSubcoreMesh(core_axis_name="c", subcore_axis_name="s")

    @functools.partial(
        pl.kernel,
        out_type=jax.ShapeDtypeStruct((total, d), x.dtype),
        mesh=mesh,
        scratch_types=[
            pltpu.VMEM((n_chunks, chunk), jnp.int32),
            pltpu.VMEM((chunk, d), x.dtype),
            pltpu.SemaphoreType.DMA,
        ],
    )
    def k(x_hbm, idx_hbm, out_hbm, idx_v, buf_v, sem):
        wid = lax.axis_index("s") * _NC + lax.axis_index("c")
        base = wid * per_w
        pltpu.sync_copy(idx_hbm.at[wid], idx_v)

        @pl.loop(0, n_chunks)
        def _(j):
            pltpu.async_copy(x_hbm.at[idx_v.at[j]], buf_v, sem).wait()
            pltpu.sync_copy(buf_v, out_hbm.at[pl.ds(base + j * chunk, chunk)])

    return k(x, idx3)


def _sc_scatter_residual(h_parts, dst3, x):
    n, d = x.shape
    dh = d // _NC
    ns, n_chunks, chunk = dst3.shape
    per_s = n_chunks * chunk
    nparts = len(h_parts)
    sub_per_part = ns // nparts
    rows_main = (n // ns) // 8 * 8
    tail_base = ns * rows_main
    tail_rows = n - tail_base
    mesh = plsc.VectorSubcoreMesh(core_axis_name="c", subcore_axis_name="s")

    @functools.partial(
        pl.kernel,
        out_type=jax.ShapeDtypeStruct((n, d), x.dtype),
        mesh=mesh,
        scratch_types=[
            pltpu.VMEM((n_chunks, chunk), jnp.int32),
            pltpu.VMEM((chunk, dh), x.dtype),
            pltpu.VMEM_SHARED((n, dh), x.dtype),
        ],
    )
    def k(*refs):
        h_refs = refs[:nparts]
        dst_hbm, x_hbm, out_hbm, idx_v, buf_v, acc_sh = refs[nparts:]
        c = lax.axis_index("c")
        s = lax.axis_index("s")
        col0 = c * dh
        r0 = s * rows_main
        pltpu.sync_copy(
            x_hbm.at[pl.ds(r0, rows_main), pl.ds(col0, dh)],
            acc_sh.at[pl.ds(r0, rows_main)],
        )
        if tail_rows:
            @pl.when(s == ns - 1)
            def _():
                pltpu.sync_copy(
                    x_hbm.at[pl.ds(tail_base, tail_rows), pl.ds(col0, dh)],
                    acc_sh.at[pl.ds(tail_base, tail_rows)],
                )
        pltpu.sync_copy(dst_hbm.at[s], idx_v)
        plsc.subcore_barrier()

        for kp in range(nparts):
            @pl.when(s // sub_per_part == kp)
            def _(kp=kp):
                @pl.loop(0, n_chunks)
                def _(j):
                    rbase = (s % sub_per_part) * per_s + j * chunk
                    pltpu.sync_copy(
                        h_refs[kp].at[pl.ds(rbase, chunk), pl.ds(col0, dh)],
                        buf_v)
                    pltpu.sync_copy(buf_v, acc_sh.at[idx_v.at[j]], add=True)

        plsc.subcore_barrier()
        pltpu.sync_copy(
            acc_sh.at[pl.ds(r0, rows_main)],
            out_hbm.at[pl.ds(r0, rows_main), pl.ds(col0, dh)],
        )
        if tail_rows:
            @pl.when(s == ns - 1)
            def _():
                pltpu.sync_copy(
                    acc_sh.at[pl.ds(tail_base, tail_rows)],
                    out_hbm.at[pl.ds(tail_base, tail_rows), pl.ds(col0, dh)],
                )

    return k(*h_parts, dst3, x)


def _tc_lstm(seq_all, steps, w_ih, w_hh, bias, blk):
    lp, d = seq_all.shape
    g = w_ih.shape[0]
    p = lp // steps
    nblk = p // blk
    prec = lax.Precision.DEFAULT
    dn = (((1,), (1,)), ((), ()))

    def body(*refs):
        s_refs = refs[:steps]
        wih_ref, whh_ref, b_ref, out_ref = refs[steps:]
        wih = wih_ref[...]
        whh = whh_ref[...]
        b = b_ref[...]
        h = None
        c = None
        for t in range(steps):
            st = s_refs[t][...]
            gates = lax.dot_general(st, wih, dn, precision=prec,
                                    preferred_element_type=jnp.float32) + b
            if h is not None:
                gates = gates + lax.dot_general(h.astype(whh.dtype), whh, dn,
                                                precision=prec,
                                                preferred_element_type=jnp.float32)
            gi = jax.nn.sigmoid(gates[:, 0 * d:1 * d])
            gf = jax.nn.sigmoid(gates[:, 1 * d:2 * d])
            gg = jnp.tanh(gates[:, 2 * d:3 * d])
            go = jax.nn.sigmoid(gates[:, 3 * d:4 * d])
            c = gi * gg if c is None else gf * c + gi * gg
            h = go * jnp.tanh(c)
        out_ref[...] = h

    seq_specs = [
        pl.BlockSpec((blk, d), lambda i, t=t: (t * nblk + i, 0))
        for t in range(steps)
    ]
    return pl.pallas_call(
        body,
        grid=(nblk,),
        in_specs=seq_specs + [
            pl.BlockSpec((g, d), lambda i: (0, 0)),
            pl.BlockSpec((g, d), lambda i: (0, 0)),
            pl.BlockSpec((1, g), lambda i: (0, 0)),
        ],
        out_specs=pl.BlockSpec((blk, d), lambda i: (i, 0)),
        out_shape=jax.ShapeDtypeStruct((p, d), jnp.float32),
    )(*([seq_all] * steps), w_ih, w_hh, bias)


def _tc_bn_relu(y, gamma, beta):
    n, d = y.shape

    def body(y_ref, g_ref, b_ref, o_ref):
        v = y_ref[...]
        mean = jnp.mean(v, axis=0, keepdims=True)
        cent = v - mean
        var = jnp.mean(cent * cent, axis=0, keepdims=True)
        scaled = cent * lax.rsqrt(var + 1e-5) * g_ref[...] + b_ref[...]
        o_ref[...] = jnp.maximum(scaled, 0.0)

    return pl.pallas_call(
        body,
        out_shape=jax.ShapeDtypeStruct((n, d), y.dtype),
    )(y, gamma.reshape(1, d), beta.reshape(1, d))


def kernel(x, paths, W_ih, W_hh, b_ih, b_hh, gamma, beta):
    n, d = x.shape
    p, l = paths.shape
    paths = paths.astype(jnp.int32)
    bias = (b_ih + b_hh).reshape(1, 4 * d).astype(jnp.float32)

    nw = _NC * _NS
    chunk = 40
    n_slices = 4
    ps = p // n_slices
    per_w = (ps * l) // nw
    h_parts = []
    for k in range(n_slices):
        pk = paths[k * ps:(k + 1) * ps]
        idx3 = pk.T.reshape(nw, per_w // chunk, chunk)
        seq_k = _sc_gather(x, idx3)
        h_parts.append(_tc_lstm(seq_k, l, W_ih, W_hh, bias, blk=2000))

    chunk2 = 80
    per_s = p // _NS
    dst3 = paths[:, l - 1].reshape(_NS, per_s // chunk2, chunk2)
    y = _sc_scatter_residual(h_parts, dst3, x)

    return _tc_bn_relu(y, gamma, beta)


# speedup vs baseline: 1.9874x; 1.1002x over previous
import functools

import jax
import jax.numpy as jnp
from jax import lax
from jax.experimental import pallas as pl
from jax.experimental.pallas import tpu as pltpu
from jax.experimental.pallas import tpu_sc as plsc

_NC = 2
_NS = 16


def _sc_gather(x, idx_segs):
    nw = idx_segs[0].shape[0]
    d = x.shape[1]
    segs = [(a.shape[1], a.shape[2]) for a in idx_segs]
    per_w = sum(nc * ch for nc, ch in segs)
    total = nw * per_w
    mesh = plsc.VectorSubcoreMesh(core_axis_name="c", subcore_axis_name="s")

    scratch = [pltpu.VMEM((nc, ch), jnp.int32) for nc, ch in segs]
    scratch += [pltpu.VMEM((ch, d), x.dtype) for _, ch in segs]
    scratch += [pltpu.SemaphoreType.DMA]

    @functools.partial(
        pl.kernel,
        out_type=jax.ShapeDtypeStruct((total, d), x.dtype),
        mesh=mesh,
        scratch_types=scratch,
    )
    def k(x_hbm, *refs):
        nseg = len(segs)
        idx_hbms = refs[:nseg]
        out_hbm = refs[nseg]
        idx_vs = refs[nseg + 1:2 * nseg + 1]
        buf_vs = refs[2 * nseg + 1:3 * nseg + 1]
        sem = refs[3 * nseg + 1]
        wid = lax.axis_index("s") * _NC + lax.axis_index("c")
        base = wid * per_w
        off = 0
        for i, (nc, ch) in enumerate(segs):
            pltpu.sync_copy(idx_hbms[i].at[wid], idx_vs[i])

            @pl.loop(0, nc)
            def _(j, i=i, ch=ch, off=off):
                pltpu.async_copy(x_hbm.at[idx_vs[i].at[j]], buf_vs[i], sem).wait()
                pltpu.sync_copy(buf_vs[i],
                                out_hbm.at[pl.ds(base + off + j * ch, ch)])

            off += nc * ch

    return k(x, *idx_segs)


def _sc_scatter_residual(h_parts, dst3, x):
    n, d = x.shape
    dh = d // _NC
    ns, n_chunks, chunk = dst3.shape
    per_s = n_chunks * chunk
    nparts = len(h_parts)
    sub_per_part = ns // nparts
    rows_main = (n // ns) // 8 * 8
    tail_base = ns * rows_main
    tail_rows = n - tail_base
    mesh = plsc.VectorSubcoreMesh(core_axis_name="c", subcore_axis_name="s")

    @functools.partial(
        pl.kernel,
        out_type=jax.ShapeDtypeStruct((n, d), x.dtype),
        mesh=mesh,
        scratch_types=[
            pltpu.VMEM((n_chunks, chunk), jnp.int32),
            pltpu.VMEM((chunk, dh), x.dtype),
            pltpu.VMEM_SHARED((n, dh), x.dtype),
        ],
    )
    def k(*refs):
        h_refs = refs[:nparts]
        dst_hbm, x_hbm, out_hbm, idx_v, buf_v, acc_sh = refs[nparts:]
        c = lax.axis_index("c")
        s = lax.axis_index("s")
        col0 = c * dh
        r0 = s * rows_main
        pltpu.sync_copy(
            x_hbm.at[pl.ds(r0, rows_main), pl.ds(col0, dh)],
            acc_sh.at[pl.ds(r0, rows_main)],
        )
        if tail_rows:
            @pl.when(s == ns - 1)
            def _():
                pltpu.sync_copy(
                    x_hbm.at[pl.ds(tail_base, tail_rows), pl.ds(col0, dh)],
                    acc_sh.at[pl.ds(tail_base, tail_rows)],
                )
        pltpu.sync_copy(dst_hbm.at[s], idx_v)
        plsc.subcore_barrier()

        for kp in range(nparts):
            @pl.when(s // sub_per_part == kp)
            def _(kp=kp):
                @pl.loop(0, n_chunks)
                def _(j):
                    rbase = (s % sub_per_part) * per_s + j * chunk
                    pltpu.sync_copy(
                        h_refs[kp].at[pl.ds(rbase, chunk), pl.ds(col0, dh)],
                        buf_v)
                    pltpu.sync_copy(buf_v, acc_sh.at[idx_v.at[j]], add=True)

        plsc.subcore_barrier()
        pltpu.sync_copy(
            acc_sh.at[pl.ds(r0, rows_main)],
            out_hbm.at[pl.ds(r0, rows_main), pl.ds(col0, dh)],
        )
        if tail_rows:
            @pl.when(s == ns - 1)
            def _():
                pltpu.sync_copy(
                    acc_sh.at[pl.ds(tail_base, tail_rows)],
                    out_hbm.at[pl.ds(tail_base, tail_rows), pl.ds(col0, dh)],
                )

    return k(*h_parts, dst3, x)


def _tc_lstm(seq_all, steps, w_ih, w_hh, bias, blk):
    lp, d = seq_all.shape
    g = w_ih.shape[0]
    p = lp // steps
    nblk = p // blk
    prec = lax.Precision.DEFAULT
    dn = (((1,), (1,)), ((), ()))

    def body(*refs):
        s_refs = refs[:steps]
        wih_ref, whh_ref, b_ref, out_ref = refs[steps:]
        wih = wih_ref[...]
        whh = whh_ref[...]
        b = b_ref[...]
        h = None
        c = None
        for t in range(steps):
            st = s_refs[t][...]
            gates = lax.dot_general(st, wih, dn, precision=prec,
                                    preferred_element_type=jnp.float32) + b
            if h is not None:
                gates = gates + lax.dot_general(h.astype(whh.dtype), whh, dn,
                                                precision=prec,
                                                preferred_element_type=jnp.float32)
            gi = jax.nn.sigmoid(gates[:, 0 * d:1 * d])
            gf = jax.nn.sigmoid(gates[:, 1 * d:2 * d])
            gg = jnp.tanh(gates[:, 2 * d:3 * d])
            go = jax.nn.sigmoid(gates[:, 3 * d:4 * d])
            c = gi * gg if c is None else gf * c + gi * gg
            h = go * jnp.tanh(c)
        out_ref[...] = h

    seq_specs = [
        pl.BlockSpec((blk, d), lambda i, t=t: (t * nblk + i, 0))
        for t in range(steps)
    ]
    return pl.pallas_call(
        body,
        grid=(nblk,),
        in_specs=seq_specs + [
            pl.BlockSpec((g, d), lambda i: (0, 0)),
            pl.BlockSpec((g, d), lambda i: (0, 0)),
            pl.BlockSpec((1, g), lambda i: (0, 0)),
        ],
        out_specs=pl.BlockSpec((blk, d), lambda i: (i, 0)),
        out_shape=jax.ShapeDtypeStruct((p, d), jnp.float32),
    )(*([seq_all] * steps), w_ih, w_hh, bias)


def _tc_bn_relu(y, gamma, beta):
    n, d = y.shape

    def body(y_ref, g_ref, b_ref, o_ref):
        v = y_ref[...]
        mean = jnp.mean(v, axis=0, keepdims=True)
        cent = v - mean
        var = jnp.mean(cent * cent, axis=0, keepdims=True)
        scaled = cent * lax.rsqrt(var + 1e-5) * g_ref[...] + b_ref[...]
        o_ref[...] = jnp.maximum(scaled, 0.0)

    return pl.pallas_call(
        body,
        out_shape=jax.ShapeDtypeStruct((n, d), y.dtype),
    )(y, gamma.reshape(1, d), beta.reshape(1, d))


def kernel(x, paths, W_ih, W_hh, b_ih, b_hh, gamma, beta):
    n, d = x.shape
    p, l = paths.shape
    paths = paths.astype(jnp.int32)
    bias = (b_ih + b_hh).reshape(1, 4 * d).astype(jnp.float32)

    nw = _NC * _NS
    chunk = 80
    n_slices = 4
    ps = p // n_slices
    per_w = (ps * l) // nw
    n_main = per_w // chunk
    tail = per_w - n_main * chunk
    h_parts = []
    for k in range(n_slices):
        pk = paths[k * ps:(k + 1) * ps]
        flat = pk.T.reshape(nw, per_w)
        segs = [flat[:, :n_main * chunk].reshape(nw, n_main, chunk)]
        if tail:
            segs.append(flat[:, n_main * chunk:].reshape(nw, 1, tail))
        seq_k = _sc_gather(x, segs)
        h_parts.append(_tc_lstm(seq_k, l, W_ih, W_hh, bias, blk=2000))

    chunk2 = 80
    per_s = p // _NS
    dst3 = paths[:, l - 1].reshape(_NS, per_s // chunk2, chunk2)
    y = _sc_scatter_residual(h_parts, dst3, x)

    return _tc_bn_relu(y, gamma, beta)


# speedup vs baseline: 2.1221x; 1.0678x over previous
import functools

import jax
import jax.numpy as jnp
from jax import lax
from jax.experimental import pallas as pl
from jax.experimental.pallas import tpu as pltpu
from jax.experimental.pallas import tpu_sc as plsc

_NC = 2
_NS = 16


def _sc_gather(x, idx_segs):
    nw = idx_segs[0].shape[0]
    d = x.shape[1]
    segs = [(a.shape[1], a.shape[2]) for a in idx_segs]
    per_w = sum(nc * ch for nc, ch in segs)
    total = nw * per_w
    mesh = plsc.VectorSubcoreMesh(core_axis_name="c", subcore_axis_name="s")

    scratch = [pltpu.VMEM((nc, ch), jnp.int32) for nc, ch in segs]
    scratch += [pltpu.VMEM((ch, d), x.dtype) for _, ch in segs]
    scratch += [pltpu.SemaphoreType.DMA]

    @functools.partial(
        pl.kernel,
        out_type=jax.ShapeDtypeStruct((total, d), x.dtype),
        mesh=mesh,
        scratch_types=scratch,
    )
    def k(x_hbm, *refs):
        nseg = len(segs)
        idx_hbms = refs[:nseg]
        out_hbm = refs[nseg]
        idx_vs = refs[nseg + 1:2 * nseg + 1]
        buf_vs = refs[2 * nseg + 1:3 * nseg + 1]
        sem = refs[3 * nseg + 1]
        wid = lax.axis_index("s") * _NC + lax.axis_index("c")
        base = wid * per_w
        off = 0
        for i, (nc, ch) in enumerate(segs):
            pltpu.sync_copy(idx_hbms[i].at[wid], idx_vs[i])

            @pl.loop(0, nc)
            def _(j, i=i, ch=ch, off=off):
                pltpu.async_copy(x_hbm.at[idx_vs[i].at[j]], buf_vs[i], sem).wait()
                pltpu.sync_copy(buf_vs[i],
                                out_hbm.at[pl.ds(base + off + j * ch, ch)])

            off += nc * ch

    return k(x, *idx_segs)


def _sc_scatter_residual(h_parts, dst_segs, init_src):
    n, d = init_src.shape
    dh = d // _NC
    ns = dst_segs[0].shape[0]
    segs = [(a.shape[1], a.shape[2]) for a in dst_segs]
    nseg = len(segs)
    per_s = sum(nc * ch for nc, ch in segs)
    nparts = len(h_parts)
    part_rows = h_parts[0].shape[0]
    sub_per_part = part_rows // per_s
    rows_main = (n // ns) // 8 * 8
    tail_base = ns * rows_main
    tail_rows = n - tail_base
    mesh = plsc.VectorSubcoreMesh(core_axis_name="c", subcore_axis_name="s")

    scratch = [pltpu.VMEM((nc, ch), jnp.int32) for nc, ch in segs]
    scratch += [pltpu.VMEM((ch, dh), init_src.dtype) for _, ch in segs]
    scratch += [pltpu.VMEM_SHARED((n, dh), init_src.dtype)]

    @functools.partial(
        pl.kernel,
        out_type=jax.ShapeDtypeStruct((n, d), init_src.dtype),
        mesh=mesh,
        scratch_types=scratch,
    )
    def k(*refs):
        h_refs = refs[:nparts]
        dst_hbms = refs[nparts:nparts + nseg]
        x_hbm = refs[nparts + nseg]
        out_hbm = refs[nparts + nseg + 1]
        idx_vs = refs[nparts + nseg + 2:nparts + nseg + 2 + nseg]
        buf_vs = refs[nparts + nseg + 2 + nseg:nparts + nseg + 2 + 2 * nseg]
        acc_sh = refs[-1]
        c = lax.axis_index("c")
        s = lax.axis_index("s")
        col0 = c * dh
        r0 = s * rows_main
        pltpu.sync_copy(
            x_hbm.at[pl.ds(r0, rows_main), pl.ds(col0, dh)],
            acc_sh.at[pl.ds(r0, rows_main)],
        )
        if tail_rows:
            @pl.when(s == ns - 1)
            def _():
                pltpu.sync_copy(
                    x_hbm.at[pl.ds(tail_base, tail_rows), pl.ds(col0, dh)],
                    acc_sh.at[pl.ds(tail_base, tail_rows)],
                )
        for i in range(nseg):
            pltpu.sync_copy(dst_hbms[i].at[s], idx_vs[i])
        plsc.subcore_barrier()

        sub_local = lax.rem(s, sub_per_part)
        for kp in range(nparts):
            @pl.when(s // sub_per_part == kp)
            def _(kp=kp):
                off = 0
                for i, (nc, ch) in enumerate(segs):
                    @pl.loop(0, nc)
                    def _(j, i=i, ch=ch, off=off):
                        rbase = sub_local * per_s + off + j * ch
                        pltpu.sync_copy(
                            h_refs[kp].at[pl.ds(rbase, ch), pl.ds(col0, dh)],
                            buf_vs[i])
                        pltpu.sync_copy(buf_vs[i], acc_sh.at[idx_vs[i].at[j]],
                                        add=True)

                    off += nc * ch

        plsc.subcore_barrier()
        pltpu.sync_copy(
            acc_sh.at[pl.ds(r0, rows_main)],
            out_hbm.at[pl.ds(r0, rows_main), pl.ds(col0, dh)],
        )
        if tail_rows:
            @pl.when(s == ns - 1)
            def _():
                pltpu.sync_copy(
                    acc_sh.at[pl.ds(tail_base, tail_rows)],
                    out_hbm.at[pl.ds(tail_base, tail_rows), pl.ds(col0, dh)],
                )

    return k(*h_parts, *dst_segs, init_src)


def _tc_lstm(seq_all, steps, w_ih, w_hh, bias, blk):
    lp, d = seq_all.shape
    g = w_ih.shape[0]
    p = lp // steps
    nblk = p // blk
    prec = lax.Precision.DEFAULT
    dn = (((1,), (1,)), ((), ()))

    def body(*refs):
        s_refs = refs[:steps]
        wih_ref, whh_ref, b_ref, out_ref = refs[steps:]
        wih = wih_ref[...]
        whh = whh_ref[...]
        b = b_ref[...]
        h = None
        c = None
        for t in range(steps):
            st = s_refs[t][...]
            gates = lax.dot_general(st, wih, dn, precision=prec,
                                    preferred_element_type=jnp.float32) + b
            if h is not None:
                gates = gates + lax.dot_general(h.astype(whh.dtype), whh, dn,
                                                precision=prec,
                                                preferred_element_type=jnp.float32)
            gi = jax.nn.sigmoid(gates[:, 0 * d:1 * d])
            gf = jax.nn.sigmoid(gates[:, 1 * d:2 * d])
            gg = jnp.tanh(gates[:, 2 * d:3 * d])
            go = jax.nn.sigmoid(gates[:, 3 * d:4 * d])
            c = gi * gg if c is None else gf * c + gi * gg
            h = go * jnp.tanh(c)
        out_ref[...] = h

    seq_specs = [
        pl.BlockSpec((blk, d), lambda i, t=t: (t * nblk + i, 0))
        for t in range(steps)
    ]
    return pl.pallas_call(
        body,
        grid=(nblk,),
        in_specs=seq_specs + [
            pl.BlockSpec((g, d), lambda i: (0, 0)),
            pl.BlockSpec((g, d), lambda i: (0, 0)),
            pl.BlockSpec((1, g), lambda i: (0, 0)),
        ],
        out_specs=pl.BlockSpec((blk, d), lambda i: (i, 0)),
        out_shape=jax.ShapeDtypeStruct((p, d), jnp.float32),
    )(*([seq_all] * steps), w_ih, w_hh, bias)


def _tc_bn_relu(y, gamma, beta):
    n, d = y.shape

    def body(y_ref, g_ref, b_ref, o_ref):
        v = y_ref[...]
        mean = jnp.mean(v, axis=0, keepdims=True)
        cent = v - mean
        var = jnp.mean(cent * cent, axis=0, keepdims=True)
        scaled = cent * lax.rsqrt(var + 1e-5) * g_ref[...] + b_ref[...]
        o_ref[...] = jnp.maximum(scaled, 0.0)

    return pl.pallas_call(
        body,
        out_shape=jax.ShapeDtypeStruct((n, d), y.dtype),
    )(y, gamma.reshape(1, d), beta.reshape(1, d))


def kernel(x, paths, W_ih, W_hh, b_ih, b_hh, gamma, beta):
    n, d = x.shape
    p, l = paths.shape
    paths = paths.astype(jnp.int32)
    bias = (b_ih + b_hh).reshape(1, 4 * d).astype(jnp.float32)

    nw = _NC * _NS
    chunk = 80
    n_slices = 4
    ps = p // n_slices
    per_w = (ps * l) // nw

    def _split(flat2d, width):
        n_main = width // chunk
        tail = width - n_main * chunk
        rows = flat2d.shape[0]
        segs = [flat2d[:, :n_main * chunk].reshape(rows, n_main, chunk)]
        if tail:
            segs.append(flat2d[:, n_main * chunk:].reshape(rows, 1, tail))
        return segs

    h_parts = []
    for k in range(n_slices):
        pk = paths[k * ps:(k + 1) * ps]
        seq_k = _sc_gather(x, _split(pk.T.reshape(nw, per_w), per_w))
        h_parts.append(_tc_lstm(seq_k, l, W_ih, W_hh, bias, blk=2000))

    dst = paths[:, l - 1]
    half = p // 2
    per_s = half // _NS
    segs_a = _split(dst[:half].reshape(_NS, per_s), per_s)
    segs_b = _split(dst[half:].reshape(_NS, per_s), per_s)
    y_a = _sc_scatter_residual(h_parts[:n_slices // 2], segs_a, x)
    y = _sc_scatter_residual(h_parts[n_slices // 2:], segs_b, y_a)

    return _tc_bn_relu(y, gamma, beta)
